# Initial kernel scaffold; baseline (speedup 1.0000x reference)
import jax
import jax.numpy as jnp
from jax.experimental import pallas as pl


def kernel(x, edge_index, batch, params):
    raise NotImplementedError("write your pallas kernel here")


# speedup vs baseline: 5.8918x; 5.8918x over previous
import functools

import jax
import jax.numpy as jnp
from jax import lax
from jax.experimental import pallas as pl
from jax.experimental.pallas import tpu as pltpu
from jax.experimental.pallas import tpu_sc as plsc

NC = 2
NS = 16
NW = NC * NS
ROWS = 128
RPC = 8
CH = ROWS * RPC

F32 = jnp.float32


def _mesh():
    return plsc.VectorSubcoreMesh(core_axis_name="c", subcore_axis_name="s",
                                  num_cores=NC, num_subcores=NS)


def _wid():
    return lax.axis_index("s") * NC + lax.axis_index("c")


def _iref(idx_row_ref):
    return idx_row_ref


def _mk_egcl_gather(EP, K, with_h):
    outs = []
    if with_h:
        outs += [jax.ShapeDtypeStruct((EP, 16), F32),
                 jax.ShapeDtypeStruct((EP, 16), F32)]
    outs += [jax.ShapeDtypeStruct((EP, 4), F32),
             jax.ShapeDtypeStruct((EP, 4), F32)]

    scratch = []
    if with_h:
        scratch += [pltpu.VMEM((CH, 16), F32), pltpu.VMEM((CH, 16), F32)]
    scratch += [pltpu.VMEM((CH, 4), F32), pltpu.VMEM((CH, 4), F32),
                pltpu.VMEM((RPC, ROWS), jnp.int32),
                pltpu.VMEM((RPC, ROWS), jnp.int32),
                pltpu.SemaphoreType.DMA]

    @functools.partial(pl.kernel, out_type=outs, mesh=_mesh(),
                       scratch_types=scratch,
                       compiler_params=pltpu.CompilerParams(
                           use_tc_tiling_on_sc=False))
    def gather_k(*refs):
        if with_h:
            (u_h, v_h, c4_h, row2, col2, ur_o, vc_o, cr_o, cc_o,
             uv, vv, crv, ccv, idxr, idxc, sem) = refs
        else:
            (c4_h, row2, col2, cr_o, cc_o,
             crv, ccv, idxr, idxc, sem) = refs
        wid = _wid()

        def chunk(i, carry):
            ci = wid * K + i
            pltpu.sync_copy(row2.at[pl.ds(ci * RPC, RPC)], idxr)
            pltpu.sync_copy(col2.at[pl.ds(ci * RPC, RPC)], idxc)
            cps = []
            for k in range(RPC):
                sl = pl.ds(k * ROWS, ROWS)
                if with_h:
                    cps.append(pltpu.async_copy(u_h.at[_iref(idxr.at[k])], uv.at[sl], sem))
                    cps.append(pltpu.async_copy(v_h.at[_iref(idxc.at[k])], vv.at[sl], sem))
                cps.append(pltpu.async_copy(c4_h.at[_iref(idxr.at[k])], crv.at[sl], sem))
                cps.append(pltpu.async_copy(c4_h.at[_iref(idxc.at[k])], ccv.at[sl], sem))
            for cp in cps:
                cp.wait()
            esl = pl.ds(ci * CH, CH)
            if with_h:
                pltpu.sync_copy(uv, ur_o.at[esl])
                pltpu.sync_copy(vv, vc_o.at[esl])
            pltpu.sync_copy(crv, cr_o.at[esl])
            pltpu.sync_copy(ccv, cc_o.at[esl])
            return carry

        lax.fori_loop(0, K, chunk, 0)

    return gather_k


def _mk_scatter(EP, K, N2, STR, W):
    outs = [jax.ShapeDtypeStruct((NC, N2, W), F32)]
    scratch = [pltpu.VMEM((CH, W), F32),
               pltpu.VMEM((RPC, ROWS), jnp.int32),
               pltpu.VMEM_SHARED((N2, W), F32),
               pltpu.SemaphoreType.DMA]

    @functools.partial(pl.kernel, out_type=outs, mesh=_mesh(),
                       scratch_types=scratch,
                       compiler_params=pltpu.CompilerParams(
                           use_tc_tiling_on_sc=False))
    def scatter_k(pm, row2, zz, o16, pmv, idxv, acc, sem):
        cid = lax.axis_index("c")
        sid = lax.axis_index("s")
        wid = sid * NC + cid
        stripe = pl.ds(sid * STR, STR)
        pltpu.sync_copy(zz.at[stripe], acc.at[stripe])
        plsc.subcore_barrier()

        def chunk(i, carry):
            ci = wid * K + i
            pltpu.sync_copy(row2.at[pl.ds(ci * RPC, RPC)], idxv)
            pltpu.sync_copy(pm.at[pl.ds(ci * CH, CH)], pmv)
            for k in range(RPC):
                sl = pl.ds(k * ROWS, ROWS)
                pltpu.sync_copy(pmv.at[sl], acc.at[_iref(idxv.at[k])], add=True)
            return carry

        lax.fori_loop(0, K, chunk, 0)
        plsc.subcore_barrier()
        pltpu.sync_copy(acc.at[stripe], o16.at[cid].at[stripe])

    return scatter_k


def _mk_gin_agg(EP, K, N2, STR):
    outs = [jax.ShapeDtypeStruct((NC, N2, 16), F32)]
    scratch = [pltpu.VMEM((CH, 16), F32),
               pltpu.VMEM((RPC, ROWS), jnp.int32),
               pltpu.VMEM((RPC, ROWS), jnp.int32),
               pltpu.VMEM_SHARED((N2, 16), F32),
               pltpu.SemaphoreType.DMA]

    @functools.partial(pl.kernel, out_type=outs, mesh=_mesh(),
                       scratch_types=scratch,
                       compiler_params=pltpu.CompilerParams(
                           use_tc_tiling_on_sc=False))
    def gin_k(h16, row2, col2, z16, o16, gv, idxr, idxc, acc16, sem):
        cid = lax.axis_index("c")
        sid = lax.axis_index("s")
        wid = sid * NC + cid
        stripe = pl.ds(sid * STR, STR)
        pltpu.sync_copy(z16.at[stripe], acc16.at[stripe])
        plsc.subcore_barrier()

        def chunk(i, carry):
            ci = wid * K + i
            pltpu.sync_copy(row2.at[pl.ds(ci * RPC, RPC)], idxr)
            pltpu.sync_copy(col2.at[pl.ds(ci * RPC, RPC)], idxc)
            cps = []
            for k in range(RPC):
                sl = pl.ds(k * ROWS, ROWS)
                cps.append(pltpu.async_copy(h16.at[_iref(idxr.at[k])], gv.at[sl], sem))
            for cp in cps:
                cp.wait()
            for k in range(RPC):
                sl = pl.ds(k * ROWS, ROWS)
                pltpu.sync_copy(gv.at[sl], acc16.at[_iref(idxc.at[k])], add=True)
            return carry

        lax.fori_loop(0, K, chunk, 0)
        plsc.subcore_barrier()
        pltpu.sync_copy(acc16.at[stripe], o16.at[cid].at[stripe])

    return gin_k


def _mk_edge_mlp(EP, BE, with_h):
    grid = (EP // BE,)
    blk16 = pl.BlockSpec((BE, 16), lambda i: (i, 0))
    blk4 = pl.BlockSpec((BE, 4), lambda i: (i, 0))
    wfull = lambda a, b: pl.BlockSpec((a, b), lambda i: (0, 0))

    def body(*refs):
        if with_h:
            (ur, vc, cr, cc, wrad, b1, w2, b2, c1w, c1b, c2w, pm_o, pt_o) = refs
        else:
            (cr, cc, wrad, b1, w2, b2, c1w, c1b, c2w, pm_o, pt_o) = refs
        d = cr[...] - cc[...]
        radial = jnp.sum(d * d, axis=1, keepdims=True)
        z = radial * wrad[...] + b1[...]
        if with_h:
            z = z + ur[...] + vc[...]
        m1 = jax.nn.silu(z)
        m = jax.nn.silu(jnp.dot(m1, w2[...],
                                preferred_element_type=F32) + b2[...])
        y = jax.nn.silu(jnp.dot(m, c1w[...],
                                preferred_element_type=F32) + c1b[...])
        phi = jnp.dot(y, c2w[...], preferred_element_type=F32)
        pm_o[...] = m
        pt_o[...] = jnp.concatenate(
            [d[:, :3] * phi, jnp.ones((BE, 1), F32)], axis=1)

    in_specs = []
    if with_h:
        in_specs += [blk16, blk16]
    in_specs += [blk4, blk4, wfull(1, 16), wfull(1, 16), wfull(16, 16),
                 wfull(1, 16), wfull(16, 16), wfull(1, 16), wfull(16, 1)]

    return pl.pallas_call(
        body,
        grid=grid,
        in_specs=in_specs,
        out_specs=[blk16, blk4],
        out_shape=[jax.ShapeDtypeStruct((EP, 16), F32),
                   jax.ShapeDtypeStruct((EP, 4), F32)],
    )


def _pad_rows(a, n_rows):
    return jnp.concatenate(
        [a, jnp.zeros((n_rows - a.shape[0], a.shape[1]), a.dtype)], axis=0)


def kernel(x, edge_index, batch, params):
    N = x.shape[0]
    E = edge_index.shape[1]
    G = 64
    K = -(-E // (NW * CH))
    EP = NW * CH * K
    N2 = NS * (-(-(N + 1) // NS))
    STR = N2 // NS

    row = edge_index[0].astype(jnp.int32)
    col = edge_index[1].astype(jnp.int32)
    padi = jnp.full((EP - E,), N, jnp.int32)
    row2 = jnp.concatenate([row, padi]).reshape(-1, ROWS)
    col2 = jnp.concatenate([col, padi]).reshape(-1, ROWS)

    z16 = jnp.zeros((N2, 16), F32)
    z4 = jnp.zeros((N2, 4), F32)

    gather_h = _mk_egcl_gather(EP, K, with_h=True)
    gather_0 = _mk_egcl_gather(EP, K, with_h=False)
    scatter16 = _mk_scatter(EP, K, N2, STR, 16)
    scatter4 = _mk_scatter(EP, K, N2, STR, 4)
    gin_agg = _mk_gin_agg(EP, K, N2, STR)
    BE = min(4096, EP)
    mlp_h = _mk_edge_mlp(EP, BE, with_h=True)
    mlp_0 = _mk_edge_mlp(EP, BE, with_h=False)

    def egcl(h, coord, p, residual):
        c4 = _pad_rows(jnp.concatenate(
            [coord, jnp.zeros((N, 1), F32)], axis=1), N2)
        we = p["e1"]["w"]
        if h is None:
            wrad = we[0:1]
            cr, cc = gather_0(c4, row2, col2)
            pm, pt = mlp_0(cr, cc, wrad, p["e1"]["b"][None, :],
                           p["e2"]["w"], p["e2"]["b"][None, :],
                           p["c1"]["w"], p["c1"]["b"][None, :],
                           p["c2"]["w"])
        else:
            hin = h.shape[1]
            u = _pad_rows(h @ we[:hin], N2)
            v = _pad_rows(h @ we[hin:2 * hin], N2)
            wrad = we[2 * hin:2 * hin + 1]
            ur, vc, cr, cc = gather_h(u, v, c4, row2, col2)
            pm, pt = mlp_h(ur, vc, cr, cc, wrad, p["e1"]["b"][None, :],
                           p["e2"]["w"], p["e2"]["b"][None, :],
                           p["c1"]["w"], p["c1"]["b"][None, :],
                           p["c2"]["w"])
        (o16,) = scatter16(pm, row2, z16)
        (o4,) = scatter4(pt, row2, z4)
        aggm = (o16[0] + o16[1])[:N]
        aggt = (o4[0] + o4[1])[:N]
        cnt = jnp.clip(aggt[:, 3:4], 1.0, None)
        coord = coord + aggt[:, :3] / cnt
        nin = aggm if h is None else jnp.concatenate([h, aggm], axis=1)
        hn = jax.nn.silu(nin @ p["n1"]["w"] + p["n1"]["b"]) @ p["n2"]["w"] \
            + p["n2"]["b"]
        if residual and h is not None:
            hn = h + hn
        return hn, coord

    def gin(h, p):
        (o16,) = gin_agg(_pad_rows(h, N2), row2, col2, z16)
        agg = (o16[0] + o16[1])[:N]
        z = h + agg
        z = z @ p["l1"]["w"] + p["l1"]["b"]
        bn = p["bn"]
        z = (z - bn["m"]) / jnp.sqrt(bn["v"] + 1e-5) * bn["g"] + bn["b"]
        z = jax.nn.relu(z)
        z = z @ p["l2"]["w"] + p["l2"]["b"]
        return z

    h, coord = egcl(None, x, params["equiv0"], False)
    h, coord = egcl(h, coord, params["equiv1"], True)
    h, coord = egcl(h, coord, params["equiv2"], True)
    h = gin(h, params["gin1"])
    h = gin(h, params["gin2"])

    cnt = jnp.clip(jax.ops.segment_sum(
        jnp.ones((N,), F32), batch, num_segments=G), 1.0, None)
    gmean = jax.ops.segment_sum(h, batch, num_segments=G) / cnt[:, None]
    gmax = jax.ops.segment_max(h, batch, num_segments=G)
    gmax = jnp.where(jnp.isfinite(gmax), gmax, 0.0)
    feat = jnp.concatenate([gmean, gmax, cnt[:, None]], axis=1)
    z = jax.nn.elu(feat @ params["cls1"]["w"] + params["cls1"]["b"], alpha=0.1)
    z = z @ params["cls2"]["w"] + params["cls2"]["b"]
    return jax.nn.softmax(z, axis=1)
